# baseline (speedup 1.0000x reference)
import functools

import jax
import jax.numpy as jnp
from jax import lax
from jax.experimental import pallas as pl
from jax.experimental.pallas import tpu as pltpu
from jax.experimental.pallas import tpu_sc as plsc

N = 10000
D = 128
E = 320000
L = 4

NC = 2
NS = 16
NW = NC * NS
B = 64
NBUF = 4
CH = 160
NCHUNK = NW * CH
CH0 = 320
CH1 = 0
SCH = 32
EPT = CH * B
E_PAD = EPT * NW
N_ACC = 10240
RPT = N_ACC // NS
BN = 1000

_mesh = plsc.VectorSubcoreMesh(core_axis_name="c", subcore_axis_name="s")


def _agg_body(h_hbm, src_hbm, dst_hbm, z_hbm, p_hbm, src_v, dst_v,
              rows0, rows1, rows2, rows3, acc_sh, sem0, sem1, sem2, sem3):
    c = lax.axis_index("c")
    s = lax.axis_index("s")
    wid = c * NS + s
    rows = (rows0, rows1, rows2, rows3)
    sems = (sem0, sem1, sem2, sem3)
    pltpu.sync_copy(z_hbm, acc_sh.at[pl.ds(s * RPT, RPT)])
    plsc.subcore_barrier()

    off = jnp.where(c == 0, s * CH0, NS * CH0 + s * CH1)
    nstages = jnp.where(c == 0, CH0 // SCH, CH1 // SCH)

    def stage(t, carry):
        base = off + t * SCH
        pltpu.sync_copy(src_hbm.at[pl.ds(base, SCH)], src_v)
        pltpu.sync_copy(dst_hbm.at[pl.ds(base, SCH)], dst_v)
        for k in range(NBUF):
            pltpu.async_copy(h_hbm.at[src_v.at[k]], rows[k], sems[k])

        def group(i, carry2):
            j = i * NBUF
            for k in range(NBUF):
                pltpu.make_async_copy(h_hbm.at[src_v.at[j + k]], rows[k],
                                      sems[k]).wait()
                pltpu.sync_copy(rows[k], acc_sh.at[dst_v.at[j + k]],
                                add=True)

                @pl.when(j + k + NBUF < SCH)
                def _():
                    pltpu.async_copy(h_hbm.at[src_v.at[j + k + NBUF]],
                                     rows[k], sems[k])

            return carry2

        lax.fori_loop(0, SCH // NBUF, group, 0)
        return carry

    lax.fori_loop(0, nstages, stage, 0)
    plsc.subcore_barrier()
    pltpu.sync_copy(acc_sh.at[pl.ds(s * RPT, RPT)],
                    p_hbm.at[pl.ds(c * N_ACC + s * RPT, RPT)])


_agg = pl.kernel(
    _agg_body,
    out_type=jax.ShapeDtypeStruct((NC * N_ACC, D), jnp.float32),
    mesh=_mesh,
    scratch_types=[
        pltpu.VMEM((SCH, B), jnp.int32),
        pltpu.VMEM((SCH, B), jnp.int32),
        pltpu.VMEM((B, D), jnp.float32),
        pltpu.VMEM((B, D), jnp.float32),
        pltpu.VMEM((B, D), jnp.float32),
        pltpu.VMEM((B, D), jnp.float32),
        pltpu.VMEM_SHARED((N_ACC, D), jnp.float32),
        pltpu.SemaphoreType.DMA,
        pltpu.SemaphoreType.DMA,
        pltpu.SemaphoreType.DMA,
        pltpu.SemaphoreType.DMA,
    ],
)


def _deg_body(dst_hbm, ones_hbm, z_hbm, pd_hbm, dst_v, ones_v, acc_sh):
    c = lax.axis_index("c")
    s = lax.axis_index("s")
    wid = c * NS + s
    pltpu.sync_copy(z_hbm, acc_sh.at[pl.ds(s * RPT, RPT)])
    pltpu.sync_copy(ones_hbm, ones_v)
    plsc.subcore_barrier()

    for block in range(CH // SCH):
        base = wid * CH + block * SCH
        pltpu.sync_copy(dst_hbm.at[pl.ds(base, SCH)], dst_v)

        def chunk(j, carry):
            pltpu.sync_copy(ones_v, acc_sh.at[dst_v.at[j]], add=True)
            return carry

        lax.fori_loop(0, SCH, chunk, 0)
    plsc.subcore_barrier()
    pltpu.sync_copy(acc_sh.at[pl.ds(s * RPT, RPT)],
                    pd_hbm.at[pl.ds(c * N_ACC + s * RPT, RPT)])


_deg = pl.kernel(
    _deg_body,
    out_type=jax.ShapeDtypeStruct((NC * N_ACC, D), jnp.float32),
    mesh=_mesh,
    scratch_types=[
        pltpu.VMEM((SCH, B), jnp.int32),
        pltpu.VMEM((B, D), jnp.float32),
        pltpu.VMEM_SHARED((N_ACC, D), jnp.float32),
    ],
)


def _inv_body(d0_ref, d1_ref, o_ref):
    o_ref[...] = 1.0 / jnp.maximum(d0_ref[...] + d1_ref[...], 1.0)


_inv = pl.pallas_call(
    _inv_body,
    out_shape=jax.ShapeDtypeStruct((N, D), jnp.float32),
    grid=(N // BN,),
    in_specs=[
        pl.BlockSpec((BN, D), lambda i: (i, 0)),
        pl.BlockSpec((BN, D), lambda i: (i, 0)),
    ],
    out_specs=pl.BlockSpec((BN, D), lambda i: (i, 0)),
)


def _dense_body(h_ref, p0_ref, p1_ref, inv_ref, wr_ref, wn_ref, b_ref, g_ref,
                be_ref, o_ref):
    h = h_ref[...]
    agg = (p0_ref[...] + p1_ref[...]) * inv_ref[...]
    out = jnp.dot(h, wr_ref[...], preferred_element_type=jnp.float32)
    out = out + jnp.dot(agg, wn_ref[...], preferred_element_type=jnp.float32)
    out = out + b_ref[...]
    mu = jnp.mean(out, axis=-1, keepdims=True)
    var = jnp.mean((out - mu) ** 2, axis=-1, keepdims=True)
    out = (out - mu) * lax.rsqrt(var + 1e-5) * g_ref[...] + be_ref[...]
    o_ref[...] = h + jnp.maximum(out, 0.0)


_dense = pl.pallas_call(
    _dense_body,
    out_shape=jax.ShapeDtypeStruct((N, D), jnp.float32),
    grid=(N // BN,),
    in_specs=[
        pl.BlockSpec((BN, D), lambda i: (i, 0)),
        pl.BlockSpec((BN, D), lambda i: (i, 0)),
        pl.BlockSpec((BN, D), lambda i: (i, 0)),
        pl.BlockSpec((BN, D), lambda i: (i, 0)),
        pl.BlockSpec((D, D), lambda i: (0, 0)),
        pl.BlockSpec((D, D), lambda i: (0, 0)),
        pl.BlockSpec((1, D), lambda i: (0, 0)),
        pl.BlockSpec((1, D), lambda i: (0, 0)),
        pl.BlockSpec((1, D), lambda i: (0, 0)),
    ],
    out_specs=pl.BlockSpec((BN, D), lambda i: (i, 0)),
)


def kernel(x, edge_index, W_root, W_neigh, b, gamma, beta):
    src = edge_index[0]
    dst = edge_index[1]
    pad_e = E_PAD - E
    src_p = jnp.concatenate(
        [src, jnp.zeros((pad_e,), jnp.int32)]).reshape(NW * CH, B)
    pad_dst = N + jnp.arange(pad_e, dtype=jnp.int32) % (N_ACC - N)
    dst_p = jnp.concatenate([dst, pad_dst]).reshape(NW * CH, B)
    zeros = jnp.zeros((RPT, D), jnp.float32)
    ones = jnp.ones((B, D), jnp.float32)

    pd = _deg(dst_p, ones, zeros)
    inv = _inv(pd[:N], pd[N_ACC:N_ACC + N])

    h = x
    for i in range(L):
        p = _agg(h, src_p, dst_p, zeros)
        h = _dense(h, p[:N], p[N_ACC:N_ACC + N], inv,
                   W_root[i], W_neigh[i],
                   b[i].reshape(1, D), gamma[i].reshape(1, D),
                   beta[i].reshape(1, D))
    return h


# speedup vs baseline: 1.2074x; 1.2074x over previous
import functools

import jax
import jax.numpy as jnp
from jax import lax
from jax.experimental import pallas as pl
from jax.experimental.pallas import tpu as pltpu
from jax.experimental.pallas import tpu_sc as plsc

N = 10000
D = 128
E = 320000
L = 4

NC = 2
NS = 16
NW = NC * NS
B = 64
NBUF = 4
CH = 160
NCHUNK = NW * CH
CH0 = 224
CH1 = 96
SCH = 32
EPT = CH * B
E_PAD = EPT * NW
N_ACC = 10240
RPT = N_ACC // NS
BN = 1000

_mesh = plsc.VectorSubcoreMesh(core_axis_name="c", subcore_axis_name="s")


def _agg_body(h_hbm, src_hbm, dst_hbm, z_hbm, p_hbm, src_v, dst_v,
              rows0, rows1, rows2, rows3, acc_sh, sem0, sem1, sem2, sem3):
    c = lax.axis_index("c")
    s = lax.axis_index("s")
    wid = c * NS + s
    rows = (rows0, rows1, rows2, rows3)
    sems = (sem0, sem1, sem2, sem3)
    pltpu.sync_copy(z_hbm, acc_sh.at[pl.ds(s * RPT, RPT)])
    plsc.subcore_barrier()

    off = jnp.where(c == 0, s * CH0, NS * CH0 + s * CH1)
    nstages = jnp.where(c == 0, CH0 // SCH, CH1 // SCH)

    def stage(t, carry):
        base = off + t * SCH
        pltpu.sync_copy(src_hbm.at[pl.ds(base, SCH)], src_v)
        pltpu.sync_copy(dst_hbm.at[pl.ds(base, SCH)], dst_v)
        for k in range(NBUF):
            pltpu.async_copy(h_hbm.at[src_v.at[k]], rows[k], sems[k])

        def group(i, carry2):
            j = i * NBUF
            for k in range(NBUF):
                pltpu.make_async_copy(h_hbm.at[src_v.at[j + k]], rows[k],
                                      sems[k]).wait()
                pltpu.sync_copy(rows[k], acc_sh.at[dst_v.at[j + k]],
                                add=True)

                @pl.when(j + k + NBUF < SCH)
                def _():
                    pltpu.async_copy(h_hbm.at[src_v.at[j + k + NBUF]],
                                     rows[k], sems[k])

            return carry2

        lax.fori_loop(0, SCH // NBUF, group, 0)
        return carry

    lax.fori_loop(0, nstages, stage, 0)
    plsc.subcore_barrier()
    pltpu.sync_copy(acc_sh.at[pl.ds(s * RPT, RPT)],
                    p_hbm.at[pl.ds(c * N_ACC + s * RPT, RPT)])


_agg = pl.kernel(
    _agg_body,
    out_type=jax.ShapeDtypeStruct((NC * N_ACC, D), jnp.float32),
    mesh=_mesh,
    scratch_types=[
        pltpu.VMEM((SCH, B), jnp.int32),
        pltpu.VMEM((SCH, B), jnp.int32),
        pltpu.VMEM((B, D), jnp.float32),
        pltpu.VMEM((B, D), jnp.float32),
        pltpu.VMEM((B, D), jnp.float32),
        pltpu.VMEM((B, D), jnp.float32),
        pltpu.VMEM_SHARED((N_ACC, D), jnp.float32),
        pltpu.SemaphoreType.DMA,
        pltpu.SemaphoreType.DMA,
        pltpu.SemaphoreType.DMA,
        pltpu.SemaphoreType.DMA,
    ],
)


def _deg_body(dst_hbm, ones_hbm, z_hbm, pd_hbm, dst_v, ones_v, acc_sh):
    c = lax.axis_index("c")
    s = lax.axis_index("s")
    wid = c * NS + s
    pltpu.sync_copy(z_hbm, acc_sh.at[pl.ds(s * RPT, RPT)])
    pltpu.sync_copy(ones_hbm, ones_v)
    plsc.subcore_barrier()

    for block in range(CH // SCH):
        base = wid * CH + block * SCH
        pltpu.sync_copy(dst_hbm.at[pl.ds(base, SCH)], dst_v)

        def chunk(j, carry):
            pltpu.sync_copy(ones_v, acc_sh.at[dst_v.at[j]], add=True)
            return carry

        lax.fori_loop(0, SCH, chunk, 0)
    plsc.subcore_barrier()
    pltpu.sync_copy(acc_sh.at[pl.ds(s * RPT, RPT)],
                    pd_hbm.at[pl.ds(c * N_ACC + s * RPT, RPT)])


_deg = pl.kernel(
    _deg_body,
    out_type=jax.ShapeDtypeStruct((NC * N_ACC, D), jnp.float32),
    mesh=_mesh,
    scratch_types=[
        pltpu.VMEM((SCH, B), jnp.int32),
        pltpu.VMEM((B, D), jnp.float32),
        pltpu.VMEM_SHARED((N_ACC, D), jnp.float32),
    ],
)


def _inv_body(d0_ref, d1_ref, o_ref):
    o_ref[...] = 1.0 / jnp.maximum(d0_ref[...] + d1_ref[...], 1.0)


_inv = pl.pallas_call(
    _inv_body,
    out_shape=jax.ShapeDtypeStruct((N, D), jnp.float32),
    grid=(N // BN,),
    in_specs=[
        pl.BlockSpec((BN, D), lambda i: (i, 0)),
        pl.BlockSpec((BN, D), lambda i: (i, 0)),
    ],
    out_specs=pl.BlockSpec((BN, D), lambda i: (i, 0)),
)


def _dense_body(h_ref, p0_ref, p1_ref, inv_ref, wr_ref, wn_ref, b_ref, g_ref,
                be_ref, o_ref):
    h = h_ref[...]
    agg = (p0_ref[...] + p1_ref[...]) * inv_ref[...]
    out = jnp.dot(h, wr_ref[...], preferred_element_type=jnp.float32)
    out = out + jnp.dot(agg, wn_ref[...], preferred_element_type=jnp.float32)
    out = out + b_ref[...]
    mu = jnp.mean(out, axis=-1, keepdims=True)
    var = jnp.mean((out - mu) ** 2, axis=-1, keepdims=True)
    out = (out - mu) * lax.rsqrt(var + 1e-5) * g_ref[...] + be_ref[...]
    o_ref[...] = h + jnp.maximum(out, 0.0)


_dense = pl.pallas_call(
    _dense_body,
    out_shape=jax.ShapeDtypeStruct((N, D), jnp.float32),
    grid=(N // BN,),
    in_specs=[
        pl.BlockSpec((BN, D), lambda i: (i, 0)),
        pl.BlockSpec((BN, D), lambda i: (i, 0)),
        pl.BlockSpec((BN, D), lambda i: (i, 0)),
        pl.BlockSpec((BN, D), lambda i: (i, 0)),
        pl.BlockSpec((D, D), lambda i: (0, 0)),
        pl.BlockSpec((D, D), lambda i: (0, 0)),
        pl.BlockSpec((1, D), lambda i: (0, 0)),
        pl.BlockSpec((1, D), lambda i: (0, 0)),
        pl.BlockSpec((1, D), lambda i: (0, 0)),
    ],
    out_specs=pl.BlockSpec((BN, D), lambda i: (i, 0)),
)


def kernel(x, edge_index, W_root, W_neigh, b, gamma, beta):
    src = edge_index[0]
    dst = edge_index[1]
    pad_e = E_PAD - E
    src_p = jnp.concatenate(
        [src, jnp.zeros((pad_e,), jnp.int32)]).reshape(NW * CH, B)
    pad_dst = N + jnp.arange(pad_e, dtype=jnp.int32) % (N_ACC - N)
    dst_p = jnp.concatenate([dst, pad_dst]).reshape(NW * CH, B)
    zeros = jnp.zeros((RPT, D), jnp.float32)
    ones = jnp.ones((B, D), jnp.float32)

    pd = _deg(dst_p, ones, zeros)
    inv = _inv(pd[:N], pd[N_ACC:N_ACC + N])

    h = x
    for i in range(L):
        p = _agg(h, src_p, dst_p, zeros)
        h = _dense(h, p[:N], p[N_ACC:N_ACC + N], inv,
                   W_root[i], W_neigh[i],
                   b[i].reshape(1, D), gamma[i].reshape(1, D),
                   beta[i].reshape(1, D))
    return h


# speedup vs baseline: 1.3404x; 1.1101x over previous
import functools

import jax
import jax.numpy as jnp
from jax import lax
from jax.experimental import pallas as pl
from jax.experimental.pallas import tpu as pltpu
from jax.experimental.pallas import tpu_sc as plsc

N = 10000
D = 128
E = 320000
L = 4

NC = 2
NS = 16
NW = NC * NS
B = 64
NBUF = 4
CH = 160
NCHUNK = NW * CH
CH0 = 288
CH1 = 32
SCH = 32
EPT = CH * B
E_PAD = EPT * NW
N_ACC = 10240
RPT = N_ACC // NS
BN = 1000

_mesh = plsc.VectorSubcoreMesh(core_axis_name="c", subcore_axis_name="s")


def _agg_body(h_hbm, src_hbm, dst_hbm, z_hbm, p_hbm, src_v, dst_v,
              rows0, rows1, rows2, rows3, acc_sh, sem0, sem1, sem2, sem3):
    c = lax.axis_index("c")
    s = lax.axis_index("s")
    wid = c * NS + s
    rows = (rows0, rows1, rows2, rows3)
    sems = (sem0, sem1, sem2, sem3)
    pltpu.sync_copy(z_hbm, acc_sh.at[pl.ds(s * RPT, RPT)])
    plsc.subcore_barrier()

    off = jnp.where(c == 0, s * CH0, NS * CH0 + s * CH1)
    nstages = jnp.where(c == 0, CH0 // SCH, CH1 // SCH)

    def stage(t, carry):
        base = off + t * SCH
        pltpu.sync_copy(src_hbm.at[pl.ds(base, SCH)], src_v)
        pltpu.sync_copy(dst_hbm.at[pl.ds(base, SCH)], dst_v)
        for k in range(NBUF):
            pltpu.async_copy(h_hbm.at[src_v.at[k]], rows[k], sems[k])

        def group(i, carry2):
            j = i * NBUF
            for k in range(NBUF):
                pltpu.make_async_copy(h_hbm.at[src_v.at[j + k]], rows[k],
                                      sems[k]).wait()
                pltpu.sync_copy(rows[k], acc_sh.at[dst_v.at[j + k]],
                                add=True)

                @pl.when(j + k + NBUF < SCH)
                def _():
                    pltpu.async_copy(h_hbm.at[src_v.at[j + k + NBUF]],
                                     rows[k], sems[k])

            return carry2

        lax.fori_loop(0, SCH // NBUF, group, 0)
        return carry

    lax.fori_loop(0, nstages, stage, 0)
    plsc.subcore_barrier()
    pltpu.sync_copy(acc_sh.at[pl.ds(s * RPT, RPT)],
                    p_hbm.at[pl.ds(c * N_ACC + s * RPT, RPT)])


_agg = pl.kernel(
    _agg_body,
    out_type=jax.ShapeDtypeStruct((NC * N_ACC, D), jnp.float32),
    mesh=_mesh,
    scratch_types=[
        pltpu.VMEM((SCH, B), jnp.int32),
        pltpu.VMEM((SCH, B), jnp.int32),
        pltpu.VMEM((B, D), jnp.float32),
        pltpu.VMEM((B, D), jnp.float32),
        pltpu.VMEM((B, D), jnp.float32),
        pltpu.VMEM((B, D), jnp.float32),
        pltpu.VMEM_SHARED((N_ACC, D), jnp.float32),
        pltpu.SemaphoreType.DMA,
        pltpu.SemaphoreType.DMA,
        pltpu.SemaphoreType.DMA,
        pltpu.SemaphoreType.DMA,
    ],
)


def _deg_body(dst_hbm, ones_hbm, z_hbm, pd_hbm, dst_v, ones_v, acc_sh):
    c = lax.axis_index("c")
    s = lax.axis_index("s")
    wid = c * NS + s
    pltpu.sync_copy(z_hbm, acc_sh.at[pl.ds(s * RPT, RPT)])
    pltpu.sync_copy(ones_hbm, ones_v)
    plsc.subcore_barrier()

    for block in range(CH // SCH):
        base = wid * CH + block * SCH
        pltpu.sync_copy(dst_hbm.at[pl.ds(base, SCH)], dst_v)

        def chunk(j, carry):
            pltpu.sync_copy(ones_v, acc_sh.at[dst_v.at[j]], add=True)
            return carry

        lax.fori_loop(0, SCH, chunk, 0)
    plsc.subcore_barrier()
    pltpu.sync_copy(acc_sh.at[pl.ds(s * RPT, RPT)],
                    pd_hbm.at[pl.ds(c * N_ACC + s * RPT, RPT)])


_deg = pl.kernel(
    _deg_body,
    out_type=jax.ShapeDtypeStruct((NC * N_ACC, D), jnp.float32),
    mesh=_mesh,
    scratch_types=[
        pltpu.VMEM((SCH, B), jnp.int32),
        pltpu.VMEM((B, D), jnp.float32),
        pltpu.VMEM_SHARED((N_ACC, D), jnp.float32),
    ],
)


def _inv_body(d0_ref, d1_ref, o_ref):
    o_ref[...] = 1.0 / jnp.maximum(d0_ref[...] + d1_ref[...], 1.0)


_inv = pl.pallas_call(
    _inv_body,
    out_shape=jax.ShapeDtypeStruct((N, D), jnp.float32),
    grid=(N // BN,),
    in_specs=[
        pl.BlockSpec((BN, D), lambda i: (i, 0)),
        pl.BlockSpec((BN, D), lambda i: (i, 0)),
    ],
    out_specs=pl.BlockSpec((BN, D), lambda i: (i, 0)),
)


def _dense_body(h_ref, p0_ref, p1_ref, inv_ref, wr_ref, wn_ref, b_ref, g_ref,
                be_ref, o_ref):
    h = h_ref[...]
    agg = (p0_ref[...] + p1_ref[...]) * inv_ref[...]
    out = jnp.dot(h, wr_ref[...], preferred_element_type=jnp.float32)
    out = out + jnp.dot(agg, wn_ref[...], preferred_element_type=jnp.float32)
    out = out + b_ref[...]
    mu = jnp.mean(out, axis=-1, keepdims=True)
    var = jnp.mean((out - mu) ** 2, axis=-1, keepdims=True)
    out = (out - mu) * lax.rsqrt(var + 1e-5) * g_ref[...] + be_ref[...]
    o_ref[...] = h + jnp.maximum(out, 0.0)


_dense = pl.pallas_call(
    _dense_body,
    out_shape=jax.ShapeDtypeStruct((N, D), jnp.float32),
    grid=(N // BN,),
    in_specs=[
        pl.BlockSpec((BN, D), lambda i: (i, 0)),
        pl.BlockSpec((BN, D), lambda i: (i, 0)),
        pl.BlockSpec((BN, D), lambda i: (i, 0)),
        pl.BlockSpec((BN, D), lambda i: (i, 0)),
        pl.BlockSpec((D, D), lambda i: (0, 0)),
        pl.BlockSpec((D, D), lambda i: (0, 0)),
        pl.BlockSpec((1, D), lambda i: (0, 0)),
        pl.BlockSpec((1, D), lambda i: (0, 0)),
        pl.BlockSpec((1, D), lambda i: (0, 0)),
    ],
    out_specs=pl.BlockSpec((BN, D), lambda i: (i, 0)),
)


def kernel(x, edge_index, W_root, W_neigh, b, gamma, beta):
    src = edge_index[0]
    dst = edge_index[1]
    pad_e = E_PAD - E
    src_p = jnp.concatenate(
        [src, jnp.zeros((pad_e,), jnp.int32)]).reshape(NW * CH, B)
    pad_dst = N + jnp.arange(pad_e, dtype=jnp.int32) % (N_ACC - N)
    dst_p = jnp.concatenate([dst, pad_dst]).reshape(NW * CH, B)
    zeros = jnp.zeros((RPT, D), jnp.float32)
    ones = jnp.ones((B, D), jnp.float32)

    pd = _deg(dst_p, ones, zeros)
    inv = _inv(pd[:N], pd[N_ACC:N_ACC + N])

    h = x
    for i in range(L):
        p = _agg(h, src_p, dst_p, zeros)
        h = _dense(h, p[:N], p[N_ACC:N_ACC + N], inv,
                   W_root[i], W_neigh[i],
                   b[i].reshape(1, D), gamma[i].reshape(1, D),
                   beta[i].reshape(1, D))
    return h


# speedup vs baseline: 2.4808x; 1.8508x over previous
import functools

import jax
import jax.numpy as jnp
from jax import lax
from jax.experimental import pallas as pl
from jax.experimental.pallas import tpu as pltpu
from jax.experimental.pallas import tpu_sc as plsc

N = 10000
D = 128
E = 320000
L = 4

NC = 2
NS = 16
NW = NC * NS
B = 64
NBUF = 4
CH = 160
NCHUNK = NW * CH
CH0 = 304
CH1 = 16
SCH = 32
EPT = CH * B
E_PAD = EPT * NW
N_ACC = 10240
RPT = N_ACC // NS
BN = 1000

_mesh = plsc.VectorSubcoreMesh(core_axis_name="c", subcore_axis_name="s")


def _agg_body(h_hbm, src_hbm, dst_hbm, z_hbm, p_hbm, src_v, dst_v,
              rows0, rows1, rows2, rows3, acc_sh, sem0, sem1, sem2, sem3):
    c = lax.axis_index("c")
    s = lax.axis_index("s")
    wid = c * NS + s
    rows = (rows0, rows1, rows2, rows3)
    sems = (sem0, sem1, sem2, sem3)
    pltpu.sync_copy(z_hbm, acc_sh.at[pl.ds(s * RPT, RPT)])
    plsc.subcore_barrier()

    off = jnp.where(c == 0, s * CH0, NS * CH0 + s * CH1)
    nstages = jnp.where(c == 0, CH0 // SCH, CH1 // SCH)

    def stage(t, carry):
        base = off + t * SCH
        pltpu.sync_copy(src_hbm.at[pl.ds(base, SCH)], src_v)
        pltpu.sync_copy(dst_hbm.at[pl.ds(base, SCH)], dst_v)
        for k in range(NBUF):
            pltpu.async_copy(h_hbm.at[src_v.at[k]], rows[k], sems[k])

        def group(i, carry2):
            j = i * NBUF
            for k in range(NBUF):
                pltpu.make_async_copy(h_hbm.at[src_v.at[j + k]], rows[k],
                                      sems[k]).wait()
                pltpu.sync_copy(rows[k], acc_sh.at[dst_v.at[j + k]],
                                add=True)

                @pl.when(j + k + NBUF < SCH)
                def _():
                    pltpu.async_copy(h_hbm.at[src_v.at[j + k + NBUF]],
                                     rows[k], sems[k])

            return carry2

        lax.fori_loop(0, SCH // NBUF, group, 0)
        return carry

    lax.fori_loop(0, nstages, stage, 0)
    plsc.subcore_barrier()
    pltpu.sync_copy(acc_sh.at[pl.ds(s * RPT, RPT)],
                    p_hbm.at[pl.ds(c * N_ACC + s * RPT, RPT)])


_agg = pl.kernel(
    _agg_body,
    out_type=jax.ShapeDtypeStruct((NC * N_ACC, D), jnp.float32),
    mesh=_mesh,
    scratch_types=[
        pltpu.VMEM((SCH, B), jnp.int32),
        pltpu.VMEM((SCH, B), jnp.int32),
        pltpu.VMEM((B, D), jnp.float32),
        pltpu.VMEM((B, D), jnp.float32),
        pltpu.VMEM((B, D), jnp.float32),
        pltpu.VMEM((B, D), jnp.float32),
        pltpu.VMEM_SHARED((N_ACC, D), jnp.float32),
        pltpu.SemaphoreType.DMA,
        pltpu.SemaphoreType.DMA,
        pltpu.SemaphoreType.DMA,
        pltpu.SemaphoreType.DMA,
    ],
)


def _deg_body(dst_hbm, ones_hbm, z_hbm, pd_hbm, dst_v, ones_v, acc_sh):
    c = lax.axis_index("c")
    s = lax.axis_index("s")
    wid = c * NS + s
    pltpu.sync_copy(z_hbm, acc_sh.at[pl.ds(s * RPT, RPT)])
    pltpu.sync_copy(ones_hbm, ones_v)
    plsc.subcore_barrier()

    for block in range(CH // SCH):
        base = wid * CH + block * SCH
        pltpu.sync_copy(dst_hbm.at[pl.ds(base, SCH)], dst_v)

        def chunk(j, carry):
            pltpu.sync_copy(ones_v, acc_sh.at[dst_v.at[j]], add=True)
            return carry

        lax.fori_loop(0, SCH, chunk, 0)
    plsc.subcore_barrier()
    pltpu.sync_copy(acc_sh.at[pl.ds(s * RPT, RPT)],
                    pd_hbm.at[pl.ds(c * N_ACC + s * RPT, RPT)])


_deg = pl.kernel(
    _deg_body,
    out_type=jax.ShapeDtypeStruct((NC * N_ACC, D), jnp.float32),
    mesh=_mesh,
    scratch_types=[
        pltpu.VMEM((SCH, B), jnp.int32),
        pltpu.VMEM((B, D), jnp.float32),
        pltpu.VMEM_SHARED((N_ACC, D), jnp.float32),
    ],
)


def _inv_body(d0_ref, d1_ref, o_ref):
    o_ref[...] = 1.0 / jnp.maximum(d0_ref[...] + d1_ref[...], 1.0)


_inv = pl.pallas_call(
    _inv_body,
    out_shape=jax.ShapeDtypeStruct((N, D), jnp.float32),
    grid=(N // BN,),
    in_specs=[
        pl.BlockSpec((BN, D), lambda i: (i, 0)),
        pl.BlockSpec((BN, D), lambda i: (i, 0)),
    ],
    out_specs=pl.BlockSpec((BN, D), lambda i: (i, 0)),
)


def _dense_body(h_ref, p0_ref, p1_ref, inv_ref, wr_ref, wn_ref, b_ref, g_ref,
                be_ref, o_ref):
    h = h_ref[...]
    agg = (p0_ref[...] + p1_ref[...]) * inv_ref[...]
    out = jnp.dot(h, wr_ref[...], preferred_element_type=jnp.float32)
    out = out + jnp.dot(agg, wn_ref[...], preferred_element_type=jnp.float32)
    out = out + b_ref[...]
    mu = jnp.mean(out, axis=-1, keepdims=True)
    var = jnp.mean((out - mu) ** 2, axis=-1, keepdims=True)
    out = (out - mu) * lax.rsqrt(var + 1e-5) * g_ref[...] + be_ref[...]
    o_ref[...] = h + jnp.maximum(out, 0.0)


_dense = pl.pallas_call(
    _dense_body,
    out_shape=jax.ShapeDtypeStruct((N, D), jnp.float32),
    grid=(N // BN,),
    in_specs=[
        pl.BlockSpec((BN, D), lambda i: (i, 0)),
        pl.BlockSpec((BN, D), lambda i: (i, 0)),
        pl.BlockSpec((BN, D), lambda i: (i, 0)),
        pl.BlockSpec((BN, D), lambda i: (i, 0)),
        pl.BlockSpec((D, D), lambda i: (0, 0)),
        pl.BlockSpec((D, D), lambda i: (0, 0)),
        pl.BlockSpec((1, D), lambda i: (0, 0)),
        pl.BlockSpec((1, D), lambda i: (0, 0)),
        pl.BlockSpec((1, D), lambda i: (0, 0)),
    ],
    out_specs=pl.BlockSpec((BN, D), lambda i: (i, 0)),
)


def kernel(x, edge_index, W_root, W_neigh, b, gamma, beta):
    src = edge_index[0]
    dst = edge_index[1]
    pad_e = E_PAD - E
    src_p = jnp.concatenate(
        [src, jnp.zeros((pad_e,), jnp.int32)]).reshape(NW * CH, B)
    pad_dst = N + jnp.arange(pad_e, dtype=jnp.int32) % (N_ACC - N)
    dst_p = jnp.concatenate([dst, pad_dst]).reshape(NW * CH, B)
    zeros = jnp.zeros((RPT, D), jnp.float32)
    ones = jnp.ones((B, D), jnp.float32)

    pd = _deg(dst_p, ones, zeros)
    inv = _inv(pd[:N], pd[N_ACC:N_ACC + N])

    h = x
    for i in range(L):
        p = _agg(h, src_p, dst_p, zeros)
        h = _dense(h, p[:N], p[N_ACC:N_ACC + N], inv,
                   W_root[i], W_neigh[i],
                   b[i].reshape(1, D), gamma[i].reshape(1, D),
                   beta[i].reshape(1, D))
    return h
